# baseline (speedup 1.0000x reference)
import jax
import jax.numpy as jnp
from jax.experimental import pallas as pl
from jax.experimental.pallas import tpu as pltpu

EPS = 1e-5


def _make_fused_kernel(n, h, w, ho, wo, L, m_valid, RC, IC):
    def _body(x_ref, w_ref, g_ref, b_ref, o_ref, y_scr, st_scr):
        i = pl.program_id(0)
        cout = o_ref.shape[3]

        @pl.when(i < n // IC)
        def _conv():
            ssum = jnp.zeros((1, cout), jnp.float32)
            sssq = jnp.zeros((1, cout), jnp.float32)
            for k in range(IC):
                xs = x_ref[k]
                cin = xs.shape[1]
                lx = 2 * w + L
                xcat = jnp.concatenate(
                    [xs[kw:kw + lx, :] for kw in range(3)], axis=1)
                acc = jnp.zeros((L, cout), jnp.float32)
                for kh in range(3):
                    off = kh * w
                    acc = acc + jnp.dot(
                        xcat[off:off + L, :], w_ref[kh],
                        preferred_element_type=jnp.float32)

                accp = jnp.concatenate(
                    [acc, jnp.zeros((ho * w - L, cout), jnp.float32)], axis=0)
                y_scr[pl.ds(i * IC + k, 1)] = accp.astype(
                    jnp.bfloat16).reshape(1, ho, w, cout)

                row = jax.lax.broadcasted_iota(jnp.int32, (L, 1), 0)
                mask = (row % w) < wo
                accm = jnp.where(mask, acc, 0.0)
                ssum = ssum + jnp.sum(accm, axis=0, keepdims=True)
                sssq = sssq + jnp.sum(accm * acc, axis=0, keepdims=True)
            sq = jnp.concatenate([ssum, sssq], axis=0)
            prev = jnp.where(i == 0, 0.0, st_scr[0:2])
            st_scr[0:2] = prev + sq

        @pl.when(i >= n // IC)
        def _bn_rows():
            j = i - n // IC
            tot = st_scr[0:1]
            tsq = st_scr[1:2]
            mean = tot / m_valid
            var = jnp.maximum(tsq / m_valid - mean * mean, 0.0)
            inv = jax.lax.rsqrt(var + EPS)
            scale = (g_ref[...] * inv).reshape(1, 1, 1, cout)
            shift = (b_ref[...] - mean * g_ref[...] * inv).reshape(
                1, 1, 1, cout)

            slab = y_scr[:, pl.ds(j * RC, RC), :wo, :]
            z = jnp.maximum(slab.astype(jnp.float32) * scale + shift, 0.0)
            o_ref[...] = jnp.transpose(z, (1, 2, 0, 3))

    return _body


def kernel(x_nchw, w_oihw, bias, gamma, beta):
    del bias
    n, cin, h, w = x_nchw.shape
    cout = w_oihw.shape[0]
    ho, wo = h - 2, w - 2
    L = ho * w - (w - wo)
    RC = next(r for r in (9, 6, 3, 2, 1) if ho % r == 0)
    IC = next(c for c in (8, 4, 2, 1) if n % c == 0)

    x_flat = jnp.transpose(x_nchw, (0, 2, 3, 1)).reshape(n, h * w, cin)

    w_taps = jnp.transpose(w_oihw, (2, 3, 1, 0)).reshape(3, 3 * cin, cout)
    g_row = gamma.reshape(1, cout)
    b_row = beta.reshape(1, cout)

    out_p = pl.pallas_call(
        _make_fused_kernel(n, h, w, ho, wo, L, float(n * ho * wo), RC, IC),
        out_shape=jax.ShapeDtypeStruct((ho, wo, n, cout), jnp.float32),
        grid=(n // IC + ho // RC,),
        in_specs=[
            pl.BlockSpec((IC, h * w, cin),
                         lambda i, _ic=IC: (jnp.minimum(i, n // _ic - 1), 0, 0)),
            pl.BlockSpec((3, 3 * cin, cout), lambda i: (0, 0, 0)),
            pl.BlockSpec((1, cout), lambda i: (0, 0)),
            pl.BlockSpec((1, cout), lambda i: (0, 0)),
        ],
        out_specs=pl.BlockSpec((RC, wo, n, cout),
                               lambda i, _ic=IC: (jnp.maximum(i - n // _ic, 0),
                                                  0, 0, 0)),
        scratch_shapes=[
            pltpu.VMEM((n, ho, w, cout), jnp.bfloat16),
            pltpu.VMEM((8, cout), jnp.float32),
        ],
        compiler_params=pltpu.CompilerParams(
            dimension_semantics=("arbitrary",)),
    )(x_flat, w_taps, g_row, b_row)

    return jnp.transpose(out_p, (2, 3, 0, 1))


# speedup vs baseline: 1.0089x; 1.0089x over previous
import jax
import jax.numpy as jnp
from jax.experimental import pallas as pl
from jax.experimental.pallas import tpu as pltpu

EPS = 1e-5


def _make_fused_kernel(n, h, w, ho, wo, L, m_valid, RC, IC):
    def _body(x_ref, w_ref, g_ref, b_ref, o_ref, y_scr, st_scr):
        i = pl.program_id(0)
        cout = o_ref.shape[3]

        @pl.when(i < n // IC)
        def _conv():
            ssum = jnp.zeros((1, cout), jnp.float32)
            sssq = jnp.zeros((1, cout), jnp.float32)
            for k in range(IC):
                xs = x_ref[k]
                cin = xs.shape[1]
                lx = 2 * w + L
                xcat = jnp.concatenate(
                    [xs[kw:kw + lx, :] for kw in range(3)], axis=1)
                acc = jnp.zeros((L, cout), jnp.float32)
                for kh in range(3):
                    off = kh * w
                    acc = acc + jnp.dot(
                        xcat[off:off + L, :], w_ref[kh],
                        preferred_element_type=jnp.float32)

                accp = jnp.concatenate(
                    [acc, jnp.zeros((ho * w - L, cout), jnp.float32)], axis=0)
                y_scr[pl.ds(i * IC + k, 1)] = accp.astype(
                    jnp.bfloat16).reshape(1, ho, w, cout)

                row = jax.lax.broadcasted_iota(jnp.int32, (L, 1), 0)
                mask = (row % w) < wo
                accm = jnp.where(mask, acc, 0.0)
                ssum = ssum + jnp.sum(accm, axis=0, keepdims=True)
                sssq = sssq + jnp.sum(accm * acc, axis=0, keepdims=True)
            sq = jnp.concatenate([ssum, sssq], axis=0)
            prev = jnp.where(i == 0, 0.0, st_scr[0:2])
            st_scr[0:2] = prev + sq

        @pl.when(i >= n // IC)
        def _bn_rows():
            j = i - n // IC
            tot = st_scr[0:1]
            tsq = st_scr[1:2]
            mean = tot / m_valid
            var = jnp.maximum(tsq / m_valid - mean * mean, 0.0)
            inv = jax.lax.rsqrt(var + EPS)
            scale = (g_ref[...] * inv).reshape(1, 1, 1, cout)
            shift = (b_ref[...] - mean * g_ref[...] * inv).reshape(
                1, 1, 1, cout)

            slab = y_scr[:, pl.ds(j * RC, RC), :wo, :]
            z = jnp.maximum(slab.astype(jnp.float32) * scale + shift, 0.0)
            o_ref[...] = jnp.transpose(z, (1, 2, 0, 3))

    return _body


def kernel(x_nchw, w_oihw, bias, gamma, beta):
    del bias
    n, cin, h, w = x_nchw.shape
    cout = w_oihw.shape[0]
    ho, wo = h - 2, w - 2
    L = ho * w - (w - wo)
    RC = next(r for r in (9, 6, 3, 2, 1) if ho % r == 0)
    IC = next(c for c in (4, 2, 1) if n % c == 0)

    x_flat = jnp.transpose(x_nchw, (0, 2, 3, 1)).reshape(n, h * w, cin)

    w_taps = jnp.transpose(w_oihw, (2, 3, 1, 0)).reshape(3, 3 * cin, cout)
    g_row = gamma.reshape(1, cout)
    b_row = beta.reshape(1, cout)

    out_p = pl.pallas_call(
        _make_fused_kernel(n, h, w, ho, wo, L, float(n * ho * wo), RC, IC),
        out_shape=jax.ShapeDtypeStruct((ho, wo, n, cout), jnp.float32),
        grid=(n // IC + ho // RC,),
        in_specs=[
            pl.BlockSpec((IC, h * w, cin),
                         lambda i, _ic=IC: (jnp.minimum(i, n // _ic - 1), 0, 0)),
            pl.BlockSpec((3, 3 * cin, cout), lambda i: (0, 0, 0)),
            pl.BlockSpec((1, cout), lambda i: (0, 0)),
            pl.BlockSpec((1, cout), lambda i: (0, 0)),
        ],
        out_specs=pl.BlockSpec((RC, wo, n, cout),
                               lambda i, _ic=IC: (jnp.maximum(i - n // _ic, 0),
                                                  0, 0, 0)),
        scratch_shapes=[
            pltpu.VMEM((n, ho, w, cout), jnp.bfloat16),
            pltpu.VMEM((8, cout), jnp.float32),
        ],
        compiler_params=pltpu.CompilerParams(
            dimension_semantics=("arbitrary",)),
    )(x_flat, w_taps, g_row, b_row)

    return jnp.transpose(out_p, (2, 3, 0, 1))
